# baseline (speedup 1.0000x reference)
import functools

import jax
import jax.numpy as jnp
from jax import lax
from jax.experimental import pallas as pl
from jax.experimental.pallas import tpu as pltpu
from jax.experimental.pallas import tpu_sc as plsc

B = 16384
D = 32
V = 1000000
NC, NS, L = 2, 16, 16
NW = NC * NS
BPW = B // NW
NG = BPW // L
M = D // 4
QPW = BPW * M
CHUNK = 128
NCH = QPW // CHUNK


def _quad_view(table):
    return (table.reshape(V // 32, 32, M, 4)
            .transpose(0, 2, 1, 3)
            .reshape(V * M, 4))


def _sc_gather(tq_u, tq_i, u, i, j):
    mesh = plsc.VectorSubcoreMesh(core_axis_name="c", subcore_axis_name="s")

    @functools.partial(
        pl.kernel,
        mesh=mesh,
        compiler_params=pltpu.CompilerParams(use_tc_tiling_on_sc=False),
        out_type=[
            jax.ShapeDtypeStruct((NW, QPW, 4), jnp.float32),
            jax.ShapeDtypeStruct((NW, QPW, 4), jnp.float32),
            jax.ShapeDtypeStruct((NW, QPW, 4), jnp.float32),
        ],
        scratch_types=[
            pltpu.VMEM((BPW,), jnp.int32),
            pltpu.VMEM((NCH, CHUNK), jnp.int32),
            pltpu.VMEM((NCH, CHUNK), jnp.int32),
            pltpu.VMEM((NCH, CHUNK), jnp.int32),
            pltpu.VMEM((QPW, 4), jnp.float32),
            pltpu.VMEM((QPW, 4), jnp.float32),
            pltpu.VMEM((QPW, 4), jnp.float32),
            pltpu.SemaphoreType.DMA,
        ],
    )
    def k(tu_hbm, ti_hbm, u_hbm, i_hbm, j_hbm, ou_hbm, op_hbm, on_hbm,
          idx_s, q_u, q_i, q_j, d_u, d_p, d_n, sem):
        wid = lax.axis_index("s") * NC + lax.axis_index("c")
        base = wid * BPW

        def fill(idx_hbm, q_ref):
            pltpu.sync_copy(idx_hbm.at[pl.ds(base, BPW)], idx_s)

            def grp(g, carry):
                v = idx_s[pl.ds(g * L, L)]
                b0 = lax.shift_left(lax.shift_right_logical(v, 5), 8)
                b = lax.bitwise_or(b0, lax.bitwise_and(v, 31))
                row8 = g // 8
                col = (g % 8) * L
                for m in range(M):
                    q_ref[m * 4 + row8, pl.ds(col, L)] = b + m * 32
                return carry

            lax.fori_loop(0, NG, grp, 0)

        fill(u_hbm, q_u)
        fill(i_hbm, q_i)
        fill(j_hbm, q_j)

        copies = []
        for table, q_ref, dst in ((tu_hbm, q_u, d_u), (ti_hbm, q_i, d_p),
                                  (ti_hbm, q_j, d_n)):
            for ch in range(NCH):
                copies.append(pltpu.async_copy(
                    table.at[q_ref.at[ch]],
                    dst.at[pl.ds(ch * CHUNK, CHUNK)], sem))
        for cp in copies:
            cp.wait()
        pltpu.sync_copy(d_u, ou_hbm.at[wid])
        pltpu.sync_copy(d_p, op_hbm.at[wid])
        pltpu.sync_copy(d_n, on_hbm.at[wid])

    return k(tq_u, tq_i, u, i, j)


def _tc_reduce(gu, gp, gn):
    def body(u_ref, p_ref, n_ref, bpr_ref, reg_ref):
        u = u_ref[...]
        p = p_ref[...]
        n = n_ref[...]
        h = u * (n - p)
        a = lax.broadcasted_iota(jnp.int32, (NW, NW * M), 0)
        bcol = lax.broadcasted_iota(jnp.int32, (NW, NW * M), 1)
        s8 = jnp.where(bcol // M == a, 1.0, 0.0).astype(jnp.float32)
        q = jax.lax.dot_general(s8, h, (((1,), (0,)), ((), ())),
                                preferred_element_type=jnp.float32)
        q1 = jnp.concatenate([q[:, 1:], q[:, :1]], axis=1)
        q2 = jnp.concatenate([q[:, 2:], q[:, :2]], axis=1)
        q3 = jnp.concatenate([q[:, 3:], q[:, :3]], axis=1)
        q4 = q + q1 + q2 + q3
        sp = jnp.maximum(q4, 0.0) + jnp.log(1.0 + jnp.exp(-jnp.abs(q4)))
        colm = lax.broadcasted_iota(jnp.int32, q4.shape, 1)
        bpr = jnp.sum(jnp.where(colm % 4 == 0, sp, 0.0))
        reg = jnp.sum(u * u + p * p + n * n) * (1.0 / B)
        bpr_ref[...] = jnp.full((8, 128), bpr, jnp.float32)
        reg_ref[...] = jnp.full((8, 128), reg, jnp.float32)

    bpr, reg = pl.pallas_call(
        body,
        out_shape=[jax.ShapeDtypeStruct((8, 128), jnp.float32),
                   jax.ShapeDtypeStruct((8, 128), jnp.float32)],
    )(gu, gp, gn)
    return bpr[0, 0], reg[0, 0]


def kernel(user_embedding, item_embedding, u, i, j):
    u = u.astype(jnp.int32)
    i = i.astype(jnp.int32)
    j = j.astype(jnp.int32)
    ou, op, on = _sc_gather(_quad_view(user_embedding),
                            _quad_view(item_embedding), u, i, j)
    shape2d = (NW * M, BPW * 4)
    return _tc_reduce(ou.reshape(shape2d), op.reshape(shape2d),
                      on.reshape(shape2d))


# speedup vs baseline: 17.8731x; 17.8731x over previous
import functools

import jax
import jax.numpy as jnp
from jax import lax
from jax.experimental import pallas as pl
from jax.experimental.pallas import tpu as pltpu
from jax.experimental.pallas import tpu_sc as plsc

B = 16384
D = 32
NC, NS, L = 2, 16, 16
NW = NC * NS
BPW = B // NW
OR = B // 4


def _sc_gather(user_embedding, item_embedding, u, i, j):
    mesh = plsc.VectorSubcoreMesh(core_axis_name="c", subcore_axis_name="s")

    @functools.partial(
        pl.kernel,
        mesh=mesh,
        out_type=[
            jax.ShapeDtypeStruct((B, D), jnp.float32),
            jax.ShapeDtypeStruct((B, D), jnp.float32),
            jax.ShapeDtypeStruct((B, D), jnp.float32),
        ],
        scratch_types=[
            pltpu.VMEM((BPW,), jnp.int32),
            pltpu.SemaphoreType.DMA,
        ],
    )
    def k(tu_hbm, ti_hbm, u_hbm, i_hbm, j_hbm, gu_hbm, gp_hbm, gn_hbm,
          idx_s, sem):
        wid = lax.axis_index("s") * NC + lax.axis_index("c")
        base = wid * BPW

        def gather(idx_hbm, table, out_hbm):
            pltpu.sync_copy(idx_hbm.at[pl.ds(base, BPW)], idx_s)

            def body(g, carry):
                v = idx_s[pl.ds(g * L, L)]
                for r in range(L):
                    idx = v[r]
                    pos = base + g * L + r
                    pltpu.async_copy(
                        table.at[pl.ds(idx, 1), :],
                        out_hbm.at[pl.ds(pos, 1), :],
                        sem)
                return carry

            lax.fori_loop(0, BPW // L, body, 0)
            pltpu.make_async_copy(
                table.at[pl.ds(0, BPW), :],
                out_hbm.at[pl.ds(base, BPW), :],
                sem).wait()

        gather(u_hbm, tu_hbm, gu_hbm)
        gather(i_hbm, ti_hbm, gp_hbm)
        gather(j_hbm, ti_hbm, gn_hbm)

    return k(user_embedding, item_embedding, u, i, j)


def _tc_reduce(gu, gp, gn):
    def body(u_ref, p_ref, n_ref, bpr_ref, reg_ref):
        un = u_ref[...]
        pn = p_ref[...]
        nn = n_ref[...]
        h = jnp.sum(un * (nn - pn), axis=1)
        sp = jnp.maximum(h, 0.0) + jnp.log(1.0 + jnp.exp(-jnp.abs(h)))
        bpr = jnp.sum(sp)
        reg = jnp.sum(un * un + pn * pn + nn * nn) * (1.0 / B)
        bpr_ref[...] = jnp.full((8, 128), bpr, jnp.float32)
        reg_ref[...] = jnp.full((8, 128), reg, jnp.float32)

    bpr, reg = pl.pallas_call(
        body,
        out_shape=[jax.ShapeDtypeStruct((8, 128), jnp.float32),
                   jax.ShapeDtypeStruct((8, 128), jnp.float32)],
    )(gu, gp, gn)
    return bpr[0, 0], reg[0, 0]


def kernel(user_embedding, item_embedding, u, i, j):
    u = u.astype(jnp.int32)
    i = i.astype(jnp.int32)
    j = j.astype(jnp.int32)
    gu, gp, gn = _sc_gather(user_embedding, item_embedding, u, i, j)
    return _tc_reduce(gu, gp, gn)


# speedup vs baseline: 39.1713x; 2.1916x over previous
import functools

import jax
import jax.numpy as jnp
from jax import lax
from jax.experimental import pallas as pl
from jax.experimental.pallas import tpu as pltpu
from jax.experimental.pallas import tpu_sc as plsc

B = 16384
D = 32
NC, NS, L = 2, 16, 16
NW = NC * NS
BPW = B // NW
OR = B // 4


def _sc_gather(user_embedding, item_embedding, u, i, j):
    mesh = plsc.VectorSubcoreMesh(core_axis_name="c", subcore_axis_name="s")

    @functools.partial(
        pl.kernel,
        mesh=mesh,
        out_type=[
            jax.ShapeDtypeStruct((B, D), jnp.float32),
            jax.ShapeDtypeStruct((B, D), jnp.float32),
            jax.ShapeDtypeStruct((B, D), jnp.float32),
        ],
        scratch_types=[
            pltpu.VMEM((BPW,), jnp.int32),
            pltpu.VMEM((BPW, D), jnp.float32),
            pltpu.SemaphoreType.DMA,
        ],
    )
    def k(tu_hbm, ti_hbm, u_hbm, i_hbm, j_hbm, gu_hbm, gp_hbm, gn_hbm,
          idx_s, stage, sem):
        wid = lax.axis_index("s") * NC + lax.axis_index("c")
        base = wid * BPW

        def gather(idx_hbm, table, out_hbm):
            pltpu.sync_copy(idx_hbm.at[pl.ds(base, BPW)], idx_s)

            def body(g, carry):
                v = idx_s[pl.ds(g * L, L)]
                for r in range(L):
                    idx = v[r]
                    pos = g * L + r
                    pltpu.async_copy(
                        table.at[pl.ds(idx, 1), :],
                        stage.at[pl.ds(pos, 1), :],
                        sem)
                return carry

            lax.fori_loop(0, BPW // L, body, 0)
            pltpu.make_async_copy(
                table.at[pl.ds(0, BPW), :], stage, sem).wait()
            pltpu.sync_copy(stage, out_hbm.at[pl.ds(base, BPW), :])

        gather(u_hbm, tu_hbm, gu_hbm)
        gather(i_hbm, ti_hbm, gp_hbm)
        gather(j_hbm, ti_hbm, gn_hbm)

    return k(user_embedding, item_embedding, u, i, j)


def _tc_reduce(gu, gp, gn):
    def body(u_ref, p_ref, n_ref, bpr_ref, reg_ref):
        un = u_ref[...]
        pn = p_ref[...]
        nn = n_ref[...]
        h = jnp.sum(un * (nn - pn), axis=1)
        sp = jnp.maximum(h, 0.0) + jnp.log(1.0 + jnp.exp(-jnp.abs(h)))
        bpr = jnp.sum(sp)
        reg = jnp.sum(un * un + pn * pn + nn * nn) * (1.0 / B)
        bpr_ref[...] = jnp.full((8, 128), bpr, jnp.float32)
        reg_ref[...] = jnp.full((8, 128), reg, jnp.float32)

    bpr, reg = pl.pallas_call(
        body,
        out_shape=[jax.ShapeDtypeStruct((8, 128), jnp.float32),
                   jax.ShapeDtypeStruct((8, 128), jnp.float32)],
    )(gu, gp, gn)
    return bpr[0, 0], reg[0, 0]


def kernel(user_embedding, item_embedding, u, i, j):
    u = u.astype(jnp.int32)
    i = i.astype(jnp.int32)
    j = j.astype(jnp.int32)
    gu, gp, gn = _sc_gather(user_embedding, item_embedding, u, i, j)
    return _tc_reduce(gu, gp, gn)
